# baseline (speedup 1.0000x reference)
import functools
import math

import jax
import jax.numpy as jnp
from jax import lax
from jax.experimental import pallas as pl
from jax.experimental.pallas import tpu as pltpu
from jax.experimental.pallas import tpu_sc as plsc

N = 10000
E = 320000
H = 2
C = 256
HC = H * C
EPS = 1e-5
E_ALL = E + N

NC = 2
NS = 16
NW = NC * NS

K = 64
CPT = math.ceil(E_ALL / (NS * K * 2)) * 2
EPT = K * CPT
E_PAD = NS * EPT
GRP = EPT // 16

KA = 80
CPTA = E // (NW * KA)
assert NW * KA * CPTA == E

_mesh = plsc.VectorSubcoreMesh(
    core_axis_name="c", subcore_axis_name="s", num_cores=NC, num_subcores=NS
)


@functools.partial(
    pl.kernel,
    out_type=jax.ShapeDtypeStruct((NC, N, 16), jnp.float32),
    mesh=_mesh,
    scratch_types=[
        pltpu.VMEM((KA,), jnp.int32),
        pltpu.VMEM((KA, 16), jnp.float32),
        pltpu.VMEM_SHARED((N, 16), jnp.float32),
    ],
)
def _sc_loop_sums(ew16, dst0, z16, out, idx_v, rows_v, acc_sh):
    cid = lax.axis_index("c")
    sid = lax.axis_index("s")
    wid = cid * NS + sid

    @pl.when(sid == 0)
    def _():
        pltpu.sync_copy(z16, acc_sh)

    plsc.subcore_barrier()

    def chunk(c, carry):
        base = wid * (KA * CPTA) + c * KA
        pltpu.sync_copy(dst0.at[pl.ds(base, KA)], idx_v)
        pltpu.sync_copy(ew16.at[pl.ds(base, KA), :], rows_v)
        pltpu.sync_copy(rows_v, acc_sh.at[idx_v], add=True)
        return carry

    lax.fori_loop(0, CPTA, chunk, 0)
    plsc.subcore_barrier()

    @pl.when(sid == 0)
    def _():
        pltpu.sync_copy(acc_sh, out.at[cid])


_ROWS = 1000
_GRID = N // _ROWS


def _tc_pre_body(h8, sums, wl, bl, wr, br, xl_o, xr_o, la_o):
    x = h8[...]
    xl_o[...] = jnp.dot(x, wl[...], preferred_element_type=jnp.float32) + bl[...]
    xr_o[...] = jnp.dot(x, wr[...], preferred_element_type=jnp.float32) + br[...]
    s = sums[0] + sums[1]
    cnt = jnp.maximum(s[:, 6:7], 1.0)
    la_o[...] = s / cnt


def _tc_pre(h8, sums, wl, bl, wr, br):
    return pl.pallas_call(
        _tc_pre_body,
        grid=(_GRID,),
        in_specs=[
            pl.BlockSpec((_ROWS, 8), lambda i: (i, 0)),
            pl.BlockSpec((NC, _ROWS, 16), lambda i: (0, i, 0)),
            pl.BlockSpec((8, HC), lambda i: (0, 0)),
            pl.BlockSpec((1, HC), lambda i: (0, 0)),
            pl.BlockSpec((8, HC), lambda i: (0, 0)),
            pl.BlockSpec((1, HC), lambda i: (0, 0)),
        ],
        out_specs=[
            pl.BlockSpec((_ROWS, HC), lambda i: (i, 0)),
            pl.BlockSpec((_ROWS, HC), lambda i: (i, 0)),
            pl.BlockSpec((_ROWS, 16), lambda i: (i, 0)),
        ],
        out_shape=[
            jax.ShapeDtypeStruct((N, HC), jnp.float32),
            jax.ShapeDtypeStruct((N, HC), jnp.float32),
            jax.ShapeDtypeStruct((N, 16), jnp.float32),
        ],
    )(h8, sums, wl, bl, wr, br)


def _gat_epilogue(acc, den, bias, scale, shift):
    h0 = jnp.concatenate([acc[0], acc[1], acc[2], acc[3]], axis=1)
    h1 = jnp.concatenate([acc[4], acc[5], acc[6], acc[7]], axis=1)
    h0 = h0 / (den[0][:, 0:1] + 1e-16)
    h1 = h1 / (den[1][:, 0:1] + 1e-16)
    y = 0.5 * (h0 + h1) + bias[...]
    return jnp.maximum(y * scale[...] + shift[...], 0.0)


def _tc_epi_body(acc, den, bias, scale, shift, wl, bl, wr, br, xl_o, xr_o):
    g = _gat_epilogue(acc, den, bias, scale, shift)
    xl_o[...] = jnp.dot(g, wl[...], preferred_element_type=jnp.float32) + bl[...]
    xr_o[...] = jnp.dot(g, wr[...], preferred_element_type=jnp.float32) + br[...]


def _tc_epi(acc, den, bias, scale, shift, wl, bl, wr, br):
    return pl.pallas_call(
        _tc_epi_body,
        grid=(_GRID,),
        in_specs=[
            pl.BlockSpec((8, _ROWS, 64), lambda i: (0, i, 0)),
            pl.BlockSpec((NC, _ROWS, 16), lambda i: (0, i, 0)),
            pl.BlockSpec((1, C), lambda i: (0, 0)),
            pl.BlockSpec((1, C), lambda i: (0, 0)),
            pl.BlockSpec((1, C), lambda i: (0, 0)),
            pl.BlockSpec((C, HC), lambda i: (0, 0)),
            pl.BlockSpec((1, HC), lambda i: (0, 0)),
            pl.BlockSpec((C, HC), lambda i: (0, 0)),
            pl.BlockSpec((1, HC), lambda i: (0, 0)),
        ],
        out_specs=[
            pl.BlockSpec((_ROWS, HC), lambda i: (i, 0)),
            pl.BlockSpec((_ROWS, HC), lambda i: (i, 0)),
        ],
        out_shape=[
            jax.ShapeDtypeStruct((N, HC), jnp.float32),
            jax.ShapeDtypeStruct((N, HC), jnp.float32),
        ],
    )(acc, den, bias, scale, shift, wl, bl, wr, br)


def _tc_head_body(acc, den, bias, scale, shift,
                  w1, b1, s1, t1, w2, b2, s2, t2, w3, b3, out_o):
    g = _gat_epilogue(acc, den, bias, scale, shift)
    x1 = jnp.dot(g, w1[...], preferred_element_type=jnp.float32) + b1[...]
    x1 = jnp.maximum(x1 * s1[...] + t1[...], 0.0)
    x2 = jnp.dot(x1, w2[...], preferred_element_type=jnp.float32) + b2[...]
    x2 = jnp.maximum(x2 * s2[...] + t2[...], 0.0)
    out_o[...] = jnp.dot(x2, w3[...], preferred_element_type=jnp.float32) + b3[...]


def _tc_head(acc, den, bias, scale, shift, w1, b1, s1, t1, w2, b2, s2, t2,
             w3, b3):
    vec = lambda: pl.BlockSpec((1, C), lambda i: (0, 0))
    mat = lambda: pl.BlockSpec((C, C), lambda i: (0, 0))
    return pl.pallas_call(
        _tc_head_body,
        grid=(_GRID,),
        in_specs=[
            pl.BlockSpec((8, _ROWS, 64), lambda i: (0, i, 0)),
            pl.BlockSpec((NC, _ROWS, 16), lambda i: (0, i, 0)),
            vec(), vec(), vec(),
            mat(), vec(), vec(), vec(),
            mat(), vec(), vec(), vec(),
            mat(), vec(),
        ],
        out_specs=[pl.BlockSpec((_ROWS, C), lambda i: (i, 0))],
        out_shape=[jax.ShapeDtypeStruct((N, C), jnp.float32)],
    )(acc, den, bias, scale, shift, w1, b1, s1, t1, w2, b2, s2, t2, w3, b3)


def _bn_fold(p):
    scale = (p['g'] / jnp.sqrt(1.0 + EPS)).reshape(1, C)
    shift = p['b'].reshape(1, C)
    return scale, shift


def kernel(h, edge_index, edge_weight, params):
    f32 = jnp.float32
    src0 = edge_index[0]
    dst0 = edge_index[1]
    loop = jnp.arange(N, dtype=src0.dtype)
    src = jnp.concatenate([src0, loop])
    dst = jnp.concatenate([dst0, loop])

    ew16 = jnp.concatenate(
        [edge_weight, jnp.ones((E, 1), f32), jnp.zeros((E, 9), f32)], axis=1)
    z16 = jnp.zeros((N, 16), f32)

    sums = _sc_loop_sums(ew16, dst0, z16)

    h8 = jnp.concatenate([h, jnp.zeros((N, 2), f32)], axis=1)
    p1 = params['conv1']
    xl, xr, la = _tc_pre(
        h8, sums,
        jnp.concatenate([p1['Wl'].T, jnp.zeros((2, HC), f32)], axis=0),
        p1['bl'].reshape(1, HC),
        jnp.concatenate([p1['Wr'].T, jnp.zeros((2, HC), f32)], axis=0),
        p1['br'].reshape(1, HC),
    )

    eattr = jnp.concatenate([edge_weight, la[:, :6]], axis=0)
    ones_den = jnp.ones((NC, N, 16), f32)

    out = None
    for li, name in enumerate(['conv1', 'conv2', 'conv3']):
        p = params[name]
        xj = xl[src].reshape(-1, H, C)
        xi = xr[dst].reshape(-1, H, C)
        ea = (eattr @ p['We'].T).reshape(-1, H, C)
        e = jax.nn.leaky_relu(xj + xi + ea, 0.2)
        alpha = (e * p['att'][None]).sum(-1)
        m = jax.ops.segment_max(alpha, dst, num_segments=N)
        ex = jnp.exp(alpha - m[dst])
        denom = jax.ops.segment_sum(ex, dst, num_segments=N)
        a = ex / (denom[dst] + 1e-16)
        agg = jax.ops.segment_sum(xj * a[..., None], dst, num_segments=N)
        acc8 = agg.reshape(N, 8, 64).transpose(1, 0, 2)

        scale, shift = _bn_fold(params['bn%d' % (li + 1)])
        bias = p['bias'].reshape(1, C)
        if name != 'conv3':
            pn = params['conv%d' % (li + 2)]
            xl, xr = _tc_epi(acc8, ones_den, bias, scale, shift,
                             pn['Wl'].T, pn['bl'].reshape(1, HC),
                             pn['Wr'].T, pn['br'].reshape(1, HC))
        else:
            s1, t1 = _bn_fold(params['fcn1'])
            s2, t2 = _bn_fold(params['fcn2'])
            out, = _tc_head(
                acc8, ones_den, bias, scale, shift,
                params['fc1']['W'].T, params['fc1']['b'].reshape(1, C), s1, t1,
                params['fc2']['W'].T, params['fc2']['b'].reshape(1, C), s2, t2,
                params['fc3']['W'].T, params['fc3']['b'].reshape(1, C),
            )
    return out
